# baseline (speedup 1.0000x reference)
import jax
import jax.numpy as jnp
from jax import lax
from jax.experimental import pallas as pl
from jax.experimental.pallas import tpu as pltpu
from jax.experimental.pallas import tpu_sc as plsc

_B = 16384
_C = 1000
_M = 1000000
_BETA = 0.9
_GAMMA = 1.7
_SUPPRESSION_EPS = 10
_K1 = 10

_R = 512
_NB = _B // _R

_NC = 2
_NS = 16
_HALF = _M // _NC
_SP = _HALF + 8
_DUMMY = _HALF
_UPT = _B // _NS
_KI = _UPT // 128
_ROWS = _B // 128


def _ce_body(logits_ref, targets_ref, loss_ref):
    x = logits_ref[...]
    t = targets_ref[0, 0, :]
    m = jnp.max(x, axis=1)
    e = jnp.exp(x - m[:, None])
    s = jnp.sum(e, axis=1)
    cols = lax.broadcasted_iota(jnp.int32, (_R, _C), 1)
    tl = jnp.sum(jnp.where(cols == t[:, None], x, 0.0), axis=1)
    loss_ref[0, 0, :] = jnp.log(s) + m - tl


def _ce_loss(logits, targets):
    targets3 = targets.reshape(_NB, 1, _R)
    loss3 = pl.pallas_call(
        _ce_body,
        grid=(_NB,),
        in_specs=[
            pl.BlockSpec((_R, _C), lambda i: (i, 0)),
            pl.BlockSpec((1, 1, _R), lambda i: (i, 0, 0)),
        ],
        out_specs=pl.BlockSpec((1, 1, _R), lambda i: (i, 0, 0)),
        out_shape=jax.ShapeDtypeStruct((_NB, 1, _R), jnp.float32),
    )(logits, targets3)
    return loss3.reshape(_B)


def _drain(copies):
    for c in copies:
        c.wait()


def _ema_body(exp_hbm, idx_hbm, loss_hbm, dp_hbm, consts_hbm, ref_hbm,
              out_loss_hbm,
              idx_v, lidx_v, lidx2_v, pos_v, w_v, gath_v, loss_v, dp_v, nl_v,
              consts_v, fixi_v, fixp_v, fixo_v, fixc_v, fixw_idx, fixw_val,
              aux_sp, n_sp, sem):
    cid = lax.axis_index("c")
    sid = lax.axis_index("s")
    half_base = pl.multiple_of(cid * _HALF, 8)

    rowbase = sid * _KI
    pltpu.sync_copy(idx_hbm.at[pl.ds(rowbase, _KI)], idx_v)
    pltpu.sync_copy(loss_hbm.at[pl.ds(rowbase, _KI)], loss_v)
    pltpu.sync_copy(dp_hbm.at[pl.ds(rowbase, _KI)], dp_v)
    pltpu.sync_copy(consts_hbm, consts_v)
    _drain([pltpu.async_copy(exp_hbm.at[idx_v.at[j]], gath_v.at[j], sem)
            for j in range(_KI)])

    @pl.when(sid == 0)
    def _():
        fixi_v[...] = jnp.full((16,), -1, jnp.int32)
        pltpu.sync_copy(fixi_v, aux_sp.at[pl.ds(0, 16)])

    a = consts_v[0, :]
    c = consts_v[1, :]
    lane = lax.broadcasted_iota(jnp.int32, (16,), 0)
    for j in range(_KI):
        base = (rowbase + j) * 128
        for i in range(128 // 16):
            sl = pl.ds(i * 16, 16)
            g = gath_v[j, sl]
            l = loss_v[j, sl]
            n = g * _BETA + l * (1.0 - _BETA)
            gath_v[j, sl] = n
            nl_v[j, sl] = (n * a - c) / dp_v[j, sl]
            li = idx_v[j, sl] - half_base
            oob = (li < 0) | (li >= _HALF)
            lidx_v[j, sl] = jnp.where(oob, _DUMMY, li)
            pos_v[j, sl] = base + i * 16 + lane

    for j in range(_KI):
        pltpu.sync_copy(gath_v.at[j], n_sp.at[pl.ds((rowbase + j) * 128, 128)])

    plsc.subcore_barrier()

    _drain([pltpu.async_copy(pos_v.at[j], aux_sp.at[lidx_v.at[j]], sem)
            for j in range(_KI)])
    plsc.subcore_barrier()

    _drain([pltpu.async_copy(aux_sp.at[lidx_v.at[j]], w_v.at[j], sem)
            for j in range(_KI)])
    for j in range(_KI):
        for i in range(128 // 16):
            sl = pl.ds(i * 16, 16)
            active = w_v[j, sl] < pos_v[j, sl]
            lidx2_v[j, sl] = jnp.where(active, lidx_v[j, sl], _DUMMY)
    _drain([pltpu.async_copy(pos_v.at[j], aux_sp.at[lidx2_v.at[j]], sem)
            for j in range(_KI)])
    plsc.subcore_barrier()

    _drain([pltpu.async_copy(aux_sp.at[lidx_v.at[j]], w_v.at[j], sem)
            for j in range(_KI)])
    for j in range(_KI):
        for i in range(128 // 16):
            sl = pl.ds(i * 16, 16)
            keep = w_v[j, sl] == pos_v[j, sl]
            lidx2_v[j, sl] = jnp.where(keep, idx_v[j, sl], half_base + lane)
    _drain([pltpu.async_copy(gath_v.at[j], ref_hbm.at[lidx2_v.at[j]], sem)
            for j in range(_KI)])
    plsc.subcore_barrier()

    @pl.when(sid == 0)
    def _():
        pltpu.sync_copy(aux_sp.at[pl.ds(0, 16)], fixi_v)
        pltpu.sync_copy(exp_hbm.at[pl.ds(half_base, 16)], fixo_v)
        w16 = fixi_v[...]
        fixp_v[...] = jnp.maximum(w16, 0)
        _drain([pltpu.async_copy(n_sp.at[fixp_v], fixc_v, sem)])
        fixed = jnp.where(w16 >= 0, fixc_v[...], fixo_v[...])
        for k in range(128 // 16):
            fixw_val[0, pl.ds(k * 16, 16)] = fixed
            fixw_idx[0, pl.ds(k * 16, 16)] = half_base + lane
        _drain([pltpu.async_copy(fixw_val.at[0], ref_hbm.at[fixw_idx.at[0]],
                                 sem)])

    @pl.when(cid == 0)
    def _():
        pltpu.sync_copy(nl_v, out_loss_hbm.at[pl.ds(rowbase, _KI)])


def _ema_kernel():
    return pl.kernel(
        _ema_body,
        out_type=jax.ShapeDtypeStruct((_ROWS, 128), jnp.float32),
        mesh=plsc.VectorSubcoreMesh(core_axis_name="c", subcore_axis_name="s",
                                    num_cores=_NC, num_subcores=_NS),
        scratch_types=[
            pltpu.VMEM((_KI, 128), jnp.int32),
            pltpu.VMEM((_KI, 128), jnp.int32),
            pltpu.VMEM((_KI, 128), jnp.int32),
            pltpu.VMEM((_KI, 128), jnp.int32),
            pltpu.VMEM((_KI, 128), jnp.int32),
            pltpu.VMEM((_KI, 128), jnp.float32),
            pltpu.VMEM((_KI, 128), jnp.float32),
            pltpu.VMEM((_KI, 128), jnp.float32),
            pltpu.VMEM((_KI, 128), jnp.float32),
            pltpu.VMEM((2, 16), jnp.float32),
            pltpu.VMEM((16,), jnp.int32),
            pltpu.VMEM((16,), jnp.int32),
            pltpu.VMEM((16,), jnp.float32),
            pltpu.VMEM((16,), jnp.float32),
            pltpu.VMEM((1, 128), jnp.int32),
            pltpu.VMEM((1, 128), jnp.float32),
            pltpu.VMEM_SHARED((_SP,), jnp.int32),
            pltpu.VMEM_SHARED((_B,), jnp.float32),
            pltpu.SemaphoreType.DMA,
        ],
    )


def kernel(logits, targets, data_parameter_minibatch, exp_avg, index_dataset,
           epoch, switch):
    loss = _ce_loss(logits, targets.astype(jnp.int32))

    es = jnp.where(epoch < _SUPPRESSION_EPS,
                   (epoch + 1) / 10.0, 1.0).astype(jnp.float32)
    bias_cor = (1.0 - jnp.power(jnp.float32(_BETA),
                                (epoch + 1))).astype(jnp.float32)
    offset = jnp.where(switch != 0, _K1 * _GAMMA, _K1).astype(jnp.float32)
    a = es / bias_cor
    c = offset * es
    consts = jnp.stack([jnp.broadcast_to(a, (16,)), jnp.broadcast_to(c, (16,))])

    idx2 = index_dataset.astype(jnp.int32).reshape(_ROWS, 128)
    loss2 = loss.reshape(_ROWS, 128)
    dp2 = data_parameter_minibatch.reshape(_ROWS, 128)

    ref = jax.new_ref(exp_avg)
    new_loss2 = _ema_kernel()(exp_avg, idx2, loss2, dp2, consts, ref)
    return new_loss2.reshape(_B), ref[...]


# speedup vs baseline: 8.7306x; 8.7306x over previous
import jax
import jax.numpy as jnp
from jax import lax
from jax.experimental import pallas as pl
from jax.experimental.pallas import tpu as pltpu
from jax.experimental.pallas import tpu_sc as plsc

_B = 16384
_C = 1000
_M = 1000000
_BETA = 0.9
_GAMMA = 1.7
_SUPPRESSION_EPS = 10
_K1 = 10

_R = 512
_NB = _B // _R

_NC = 2
_NS = 16
_NVIC = 2048
_SP = _M + _NVIC
_UPT = _B // _NS
_KI = _UPT // 128
_ROWS = _B // 128


def _ce_body(logits_ref, targets_ref, loss_ref):
    x = logits_ref[...]
    t = targets_ref[0, 0, :]
    m = jnp.max(x, axis=1)
    e = jnp.exp(x - m[:, None])
    s = jnp.sum(e, axis=1)
    cols = lax.broadcasted_iota(jnp.int32, (_R, _C), 1)
    tl = jnp.sum(jnp.where(cols == t[:, None], x, 0.0), axis=1)
    loss_ref[0, 0, :] = jnp.log(s) + m - tl


def _ce_loss(logits, targets):
    targets3 = targets.reshape(_NB, 1, _R)
    loss3 = pl.pallas_call(
        _ce_body,
        grid=(_NB,),
        in_specs=[
            pl.BlockSpec((_R, _C), lambda i: (i, 0)),
            pl.BlockSpec((1, 1, _R), lambda i: (i, 0, 0)),
        ],
        out_specs=pl.BlockSpec((1, 1, _R), lambda i: (i, 0, 0)),
        out_shape=jax.ShapeDtypeStruct((_NB, 1, _R), jnp.float32),
    )(logits, targets3)
    return loss3.reshape(_B)


def _drain(copies):
    for c in copies:
        c.wait()


def _ema_body(exp_hbm, idx_hbm, loss_hbm, dp_hbm, consts_hbm, ref_hbm,
              out_loss_hbm,
              idx_v, lidx2_v, pos_v, w_v, gath_v, loss_v, dp_v, nl_v, vv_v,
              consts_v, aux_sp, n_sp, sem):
    cid = lax.axis_index("c")
    sid = lax.axis_index("s")

    rowbase = sid * _KI
    pltpu.sync_copy(idx_hbm.at[pl.ds(rowbase, _KI)], idx_v)
    pltpu.sync_copy(loss_hbm.at[pl.ds(rowbase, _KI)], loss_v)
    pltpu.sync_copy(dp_hbm.at[pl.ds(rowbase, _KI)], dp_v)
    pltpu.sync_copy(consts_hbm, consts_v)
    _drain([pltpu.async_copy(exp_hbm.at[idx_v.at[j]], gath_v.at[j], sem)
            for j in range(_KI)])

    a = consts_v[0, :]
    c = consts_v[1, :]
    lane = lax.broadcasted_iota(jnp.int32, (16,), 0)
    for j in range(_KI):
        base = (rowbase + j) * 128
        for i in range(128 // 16):
            sl = pl.ds(i * 16, 16)
            g = gath_v[j, sl]
            l = loss_v[j, sl]
            n = g * _BETA + l * (1.0 - _BETA)
            gath_v[j, sl] = n
            nl_v[j, sl] = (n * a - c) / dp_v[j, sl]
            pos_v[j, sl] = base + i * 16 + lane

    for j in range(_KI):
        pltpu.sync_copy(gath_v.at[j], n_sp.at[pl.ds((rowbase + j) * 128, 128)])

    _drain([pltpu.async_copy(pos_v.at[j], aux_sp.at[idx_v.at[j]], sem)
            for j in range(_KI)])
    plsc.subcore_barrier()

    _drain([pltpu.async_copy(aux_sp.at[idx_v.at[j]], w_v.at[j], sem)
            for j in range(_KI)])
    for j in range(_KI):
        for i in range(128 // 16):
            sl = pl.ds(i * 16, 16)
            active = w_v[j, sl] < pos_v[j, sl]
            dummy = _M + (pos_v[j, sl] & (_NVIC - 1))
            lidx2_v[j, sl] = jnp.where(active, idx_v[j, sl], dummy)
    _drain([pltpu.async_copy(pos_v.at[j], aux_sp.at[lidx2_v.at[j]], sem)
            for j in range(_KI)])
    plsc.subcore_barrier()

    _drain([pltpu.async_copy(aux_sp.at[idx_v.at[j]], w_v.at[j], sem)
            for j in range(_KI)])
    _drain([pltpu.async_copy(n_sp.at[w_v.at[j]], vv_v.at[j], sem)
            for j in range(_KI)])
    _drain([pltpu.async_copy(vv_v.at[j], ref_hbm.at[idx_v.at[j]], sem)
            for j in range(_KI)])

    @pl.when(cid == 0)
    def _():
        pltpu.sync_copy(nl_v, out_loss_hbm.at[pl.ds(rowbase, _KI)])


def _ema_kernel():
    return pl.kernel(
        _ema_body,
        out_type=jax.ShapeDtypeStruct((_ROWS, 128), jnp.float32),
        mesh=plsc.VectorSubcoreMesh(core_axis_name="c", subcore_axis_name="s",
                                    num_cores=_NC, num_subcores=_NS),
        scratch_types=[
            pltpu.VMEM((_KI, 128), jnp.int32),
            pltpu.VMEM((_KI, 128), jnp.int32),
            pltpu.VMEM((_KI, 128), jnp.int32),
            pltpu.VMEM((_KI, 128), jnp.int32),
            pltpu.VMEM((_KI, 128), jnp.float32),
            pltpu.VMEM((_KI, 128), jnp.float32),
            pltpu.VMEM((_KI, 128), jnp.float32),
            pltpu.VMEM((_KI, 128), jnp.float32),
            pltpu.VMEM((_KI, 128), jnp.float32),
            pltpu.VMEM((2, 16), jnp.float32),
            pltpu.VMEM_SHARED((_SP,), jnp.int32),
            pltpu.VMEM_SHARED((_B,), jnp.float32),
            pltpu.SemaphoreType.DMA,
        ],
    )


def kernel(logits, targets, data_parameter_minibatch, exp_avg, index_dataset,
           epoch, switch):
    loss = _ce_loss(logits, targets.astype(jnp.int32))

    es = jnp.where(epoch < _SUPPRESSION_EPS,
                   (epoch + 1) / 10.0, 1.0).astype(jnp.float32)
    bias_cor = (1.0 - jnp.power(jnp.float32(_BETA),
                                (epoch + 1))).astype(jnp.float32)
    offset = jnp.where(switch != 0, _K1 * _GAMMA, _K1).astype(jnp.float32)
    a = es / bias_cor
    c = offset * es
    consts = jnp.stack([jnp.broadcast_to(a, (16,)), jnp.broadcast_to(c, (16,))])

    idx2 = index_dataset.astype(jnp.int32).reshape(_ROWS, 128)
    loss2 = loss.reshape(_ROWS, 128)
    dp2 = data_parameter_minibatch.reshape(_ROWS, 128)

    ref = jax.new_ref(exp_avg)
    new_loss2 = _ema_kernel()(exp_avg, idx2, loss2, dp2, consts, ref)
    return new_loss2.reshape(_B), ref[...]


# speedup vs baseline: 8.8139x; 1.0095x over previous
import jax
import jax.numpy as jnp
from jax import lax
from jax.experimental import pallas as pl
from jax.experimental.pallas import tpu as pltpu
from jax.experimental.pallas import tpu_sc as plsc

_B = 16384
_C = 1000
_M = 1000000
_BETA = 0.9
_GAMMA = 1.7
_SUPPRESSION_EPS = 10
_K1 = 10

_R = 512
_NB = _B // _R

_NC = 2
_NS = 16
_NVIC = 2048
_SP = _M + _NVIC
_UPT = _B // _NS
_KI = _UPT // 128
_ROWS = _B // 128


def _ce_body(logits_ref, targets_ref, loss_ref):
    x = logits_ref[...]
    t = targets_ref[0, 0, :]
    m = jnp.max(x, axis=1)
    e = jnp.exp(x - m[:, None])
    s = jnp.sum(e, axis=1)
    cols = lax.broadcasted_iota(jnp.int32, (_R, _C), 1)
    tl = jnp.sum(jnp.where(cols == t[:, None], x, 0.0), axis=1)
    loss_ref[0, 0, :] = jnp.log(s) + m - tl


def _ce_loss(logits, targets):
    targets3 = targets.reshape(_NB, 1, _R)
    loss3 = pl.pallas_call(
        _ce_body,
        grid=(_NB,),
        in_specs=[
            pl.BlockSpec((_R, _C), lambda i: (i, 0)),
            pl.BlockSpec((1, 1, _R), lambda i: (i, 0, 0)),
        ],
        out_specs=pl.BlockSpec((1, 1, _R), lambda i: (i, 0, 0)),
        out_shape=jax.ShapeDtypeStruct((_NB, 1, _R), jnp.float32),
    )(logits, targets3)
    return loss3.reshape(_B)


def _drain(copies):
    for c in copies:
        c.wait()


def _ema_body(exp_hbm, idx_hbm, loss_hbm, dp_hbm, consts_hbm, ref_hbm,
              out_loss_hbm,
              idx_v, lidx2_v, pos_v, w_v, gath_v, loss_v, dp_v, nl_v, vv_v,
              consts_v, aux_sp, n_sp, sem):
    cid = lax.axis_index("c")
    sid = lax.axis_index("s")

    rowbase = sid * _KI
    c1 = pltpu.async_copy(idx_hbm.at[pl.ds(rowbase, _KI)], idx_v, sem)
    c2 = pltpu.async_copy(loss_hbm.at[pl.ds(rowbase, _KI)], loss_v, sem)
    c3 = pltpu.async_copy(dp_hbm.at[pl.ds(rowbase, _KI)], dp_v, sem)
    c4 = pltpu.async_copy(consts_hbm, consts_v, sem)
    _drain([c1, c2, c3, c4])
    _drain([pltpu.async_copy(exp_hbm.at[idx_v.at[j]], gath_v.at[j], sem)
            for j in range(_KI)])

    a = consts_v[0, :]
    c = consts_v[1, :]
    lane = lax.broadcasted_iota(jnp.int32, (16,), 0)
    for j in range(_KI):
        base = (rowbase + j) * 128
        for i in range(128 // 16):
            sl = pl.ds(i * 16, 16)
            g = gath_v[j, sl]
            l = loss_v[j, sl]
            n = g * _BETA + l * (1.0 - _BETA)
            gath_v[j, sl] = n
            nl_v[j, sl] = (n * a - c) / dp_v[j, sl]
            pos_v[j, sl] = base + i * 16 + lane

    _drain([pltpu.async_copy(gath_v.at[j],
                             n_sp.at[pl.ds((rowbase + j) * 128, 128)], sem)
            for j in range(_KI)])

    _drain([pltpu.async_copy(pos_v.at[j], aux_sp.at[idx_v.at[j]], sem)
            for j in range(_KI)])
    plsc.subcore_barrier()

    _drain([pltpu.async_copy(aux_sp.at[idx_v.at[j]], w_v.at[j], sem)
            for j in range(_KI)])
    for j in range(_KI):
        for i in range(128 // 16):
            sl = pl.ds(i * 16, 16)
            active = w_v[j, sl] < pos_v[j, sl]
            dummy = _M + (pos_v[j, sl] & (_NVIC - 1))
            lidx2_v[j, sl] = jnp.where(active, idx_v[j, sl], dummy)
    _drain([pltpu.async_copy(pos_v.at[j], aux_sp.at[lidx2_v.at[j]], sem)
            for j in range(_KI)])
    plsc.subcore_barrier()

    _drain([pltpu.async_copy(aux_sp.at[idx_v.at[j]], w_v.at[j], sem)
            for j in range(_KI)])
    _drain([pltpu.async_copy(n_sp.at[w_v.at[j]], vv_v.at[j], sem)
            for j in range(_KI)])
    _drain([pltpu.async_copy(vv_v.at[j], ref_hbm.at[idx_v.at[j]], sem)
            for j in range(_KI)])

    @pl.when(cid == 0)
    def _():
        pltpu.sync_copy(nl_v, out_loss_hbm.at[pl.ds(rowbase, _KI)])


def _ema_kernel():
    return pl.kernel(
        _ema_body,
        out_type=jax.ShapeDtypeStruct((_ROWS, 128), jnp.float32),
        mesh=plsc.VectorSubcoreMesh(core_axis_name="c", subcore_axis_name="s",
                                    num_cores=_NC, num_subcores=_NS),
        scratch_types=[
            pltpu.VMEM((_KI, 128), jnp.int32),
            pltpu.VMEM((_KI, 128), jnp.int32),
            pltpu.VMEM((_KI, 128), jnp.int32),
            pltpu.VMEM((_KI, 128), jnp.int32),
            pltpu.VMEM((_KI, 128), jnp.float32),
            pltpu.VMEM((_KI, 128), jnp.float32),
            pltpu.VMEM((_KI, 128), jnp.float32),
            pltpu.VMEM((_KI, 128), jnp.float32),
            pltpu.VMEM((_KI, 128), jnp.float32),
            pltpu.VMEM((2, 16), jnp.float32),
            pltpu.VMEM_SHARED((_SP,), jnp.int32),
            pltpu.VMEM_SHARED((_B,), jnp.float32),
            pltpu.SemaphoreType.DMA,
        ],
    )


def kernel(logits, targets, data_parameter_minibatch, exp_avg, index_dataset,
           epoch, switch):
    loss = _ce_loss(logits, targets.astype(jnp.int32))

    es = jnp.where(epoch < _SUPPRESSION_EPS,
                   (epoch + 1) / 10.0, 1.0).astype(jnp.float32)
    bias_cor = (1.0 - jnp.power(jnp.float32(_BETA),
                                (epoch + 1))).astype(jnp.float32)
    offset = jnp.where(switch != 0, _K1 * _GAMMA, _K1).astype(jnp.float32)
    a = es / bias_cor
    c = offset * es
    consts = jnp.stack([jnp.broadcast_to(a, (16,)), jnp.broadcast_to(c, (16,))])

    idx2 = index_dataset.astype(jnp.int32).reshape(_ROWS, 128)
    loss2 = loss.reshape(_ROWS, 128)
    dp2 = data_parameter_minibatch.reshape(_ROWS, 128)

    ref = jax.new_ref(exp_avg)
    new_loss2 = _ema_kernel()(exp_avg, idx2, loss2, dp2, consts, ref)
    return new_loss2.reshape(_B), ref[...]


# speedup vs baseline: 9.3025x; 1.0554x over previous
import jax
import jax.numpy as jnp
from jax import lax
from jax.experimental import pallas as pl
from jax.experimental.pallas import tpu as pltpu
from jax.experimental.pallas import tpu_sc as plsc

_B = 16384
_C = 1000
_M = 1000000
_BETA = 0.9
_GAMMA = 1.7
_SUPPRESSION_EPS = 10
_K1 = 10

_R = 1024
_NB = _B // _R

_NC = 2
_NS = 16
_NVIC = 2048
_SP = _M + _NVIC
_UPT = _B // _NS
_KI = _UPT // 128
_ROWS = _B // 128


def _ce_body(logits_ref, targets_ref, loss_ref):
    x = logits_ref[...]
    t = targets_ref[0, 0, :]
    m = jnp.max(x, axis=1)
    e = jnp.exp(x - m[:, None])
    s = jnp.sum(e, axis=1)
    cols = lax.broadcasted_iota(jnp.int32, (_R, _C), 1)
    tl = jnp.sum(jnp.where(cols == t[:, None], x, 0.0), axis=1)
    loss_ref[0, 0, :] = jnp.log(s) + m - tl


def _ce_loss(logits, targets):
    targets3 = targets.reshape(_NB, 1, _R)
    loss3 = pl.pallas_call(
        _ce_body,
        grid=(_NB,),
        in_specs=[
            pl.BlockSpec((_R, _C), lambda i: (i, 0)),
            pl.BlockSpec((1, 1, _R), lambda i: (i, 0, 0)),
        ],
        out_specs=pl.BlockSpec((1, 1, _R), lambda i: (i, 0, 0)),
        out_shape=jax.ShapeDtypeStruct((_NB, 1, _R), jnp.float32),
    )(logits, targets3)
    return loss3.reshape(_B)


def _drain(copies):
    for c in copies:
        c.wait()


def _ema_body(exp_hbm, idx_hbm, loss_hbm, dp_hbm, consts_hbm, ref_hbm,
              out_loss_hbm,
              idx_v, lidx2_v, pos_v, w_v, gath_v, loss_v, dp_v, nl_v, vv_v,
              consts_v, aux_sp, n_sp, sem):
    cid = lax.axis_index("c")
    sid = lax.axis_index("s")

    rowbase = sid * _KI
    c1 = pltpu.async_copy(idx_hbm.at[pl.ds(rowbase, _KI)], idx_v, sem)
    c2 = pltpu.async_copy(loss_hbm.at[pl.ds(rowbase, _KI)], loss_v, sem)
    c3 = pltpu.async_copy(dp_hbm.at[pl.ds(rowbase, _KI)], dp_v, sem)
    c4 = pltpu.async_copy(consts_hbm, consts_v, sem)
    _drain([c1, c2, c3, c4])
    _drain([pltpu.async_copy(exp_hbm.at[idx_v.at[j]], gath_v.at[j], sem)
            for j in range(_KI)])

    a = consts_v[0, :]
    c = consts_v[1, :]
    lane = lax.broadcasted_iota(jnp.int32, (16,), 0)
    for j in range(_KI):
        base = (rowbase + j) * 128
        for i in range(128 // 16):
            sl = pl.ds(i * 16, 16)
            g = gath_v[j, sl]
            l = loss_v[j, sl]
            n = g * _BETA + l * (1.0 - _BETA)
            gath_v[j, sl] = n
            nl_v[j, sl] = (n * a - c) / dp_v[j, sl]
            pos_v[j, sl] = base + i * 16 + lane

    _drain([pltpu.async_copy(gath_v.at[j],
                             n_sp.at[pl.ds((rowbase + j) * 128, 128)], sem)
            for j in range(_KI)])

    _drain([pltpu.async_copy(pos_v.at[j], aux_sp.at[idx_v.at[j]], sem)
            for j in range(_KI)])
    plsc.subcore_barrier()

    _drain([pltpu.async_copy(aux_sp.at[idx_v.at[j]], w_v.at[j], sem)
            for j in range(_KI)])
    for j in range(_KI):
        for i in range(128 // 16):
            sl = pl.ds(i * 16, 16)
            active = w_v[j, sl] < pos_v[j, sl]
            dummy = _M + (pos_v[j, sl] & (_NVIC - 1))
            lidx2_v[j, sl] = jnp.where(active, idx_v[j, sl], dummy)
    _drain([pltpu.async_copy(pos_v.at[j], aux_sp.at[lidx2_v.at[j]], sem)
            for j in range(_KI)])
    plsc.subcore_barrier()

    _drain([pltpu.async_copy(aux_sp.at[idx_v.at[j]], w_v.at[j], sem)
            for j in range(_KI)])
    _drain([pltpu.async_copy(n_sp.at[w_v.at[j]], vv_v.at[j], sem)
            for j in range(_KI)])
    _drain([pltpu.async_copy(vv_v.at[j], ref_hbm.at[idx_v.at[j]], sem)
            for j in range(_KI)])

    @pl.when(cid == 0)
    def _():
        pltpu.sync_copy(nl_v, out_loss_hbm.at[pl.ds(rowbase, _KI)])


def _ema_kernel():
    return pl.kernel(
        _ema_body,
        out_type=jax.ShapeDtypeStruct((_ROWS, 128), jnp.float32),
        mesh=plsc.VectorSubcoreMesh(core_axis_name="c", subcore_axis_name="s",
                                    num_cores=_NC, num_subcores=_NS),
        scratch_types=[
            pltpu.VMEM((_KI, 128), jnp.int32),
            pltpu.VMEM((_KI, 128), jnp.int32),
            pltpu.VMEM((_KI, 128), jnp.int32),
            pltpu.VMEM((_KI, 128), jnp.int32),
            pltpu.VMEM((_KI, 128), jnp.float32),
            pltpu.VMEM((_KI, 128), jnp.float32),
            pltpu.VMEM((_KI, 128), jnp.float32),
            pltpu.VMEM((_KI, 128), jnp.float32),
            pltpu.VMEM((_KI, 128), jnp.float32),
            pltpu.VMEM((2, 16), jnp.float32),
            pltpu.VMEM_SHARED((_SP,), jnp.int32),
            pltpu.VMEM_SHARED((_B,), jnp.float32),
            pltpu.SemaphoreType.DMA,
        ],
    )


def kernel(logits, targets, data_parameter_minibatch, exp_avg, index_dataset,
           epoch, switch):
    loss = _ce_loss(logits, targets.astype(jnp.int32))

    es = jnp.where(epoch < _SUPPRESSION_EPS,
                   (epoch + 1) / 10.0, 1.0).astype(jnp.float32)
    bias_cor = (1.0 - jnp.power(jnp.float32(_BETA),
                                (epoch + 1))).astype(jnp.float32)
    offset = jnp.where(switch != 0, _K1 * _GAMMA, _K1).astype(jnp.float32)
    a = es / bias_cor
    c = offset * es
    consts = jnp.stack([jnp.broadcast_to(a, (16,)), jnp.broadcast_to(c, (16,))])

    idx2 = index_dataset.astype(jnp.int32).reshape(_ROWS, 128)
    loss2 = loss.reshape(_ROWS, 128)
    dp2 = data_parameter_minibatch.reshape(_ROWS, 128)

    ref = jax.new_ref(exp_avg)
    new_loss2 = _ema_kernel()(exp_avg, idx2, loss2, dp2, consts, ref)
    return new_loss2.reshape(_B), ref[...]


# speedup vs baseline: 9.4781x; 1.0189x over previous
import jax
import jax.numpy as jnp
from jax import lax
from jax.experimental import pallas as pl
from jax.experimental.pallas import tpu as pltpu
from jax.experimental.pallas import tpu_sc as plsc

_B = 16384
_C = 1000
_M = 1000000
_BETA = 0.9
_GAMMA = 1.7
_SUPPRESSION_EPS = 10
_K1 = 10

_R = 2048
_NB = _B // _R

_NC = 2
_NS = 16
_NVIC = 2048
_SP = _M + _NVIC
_UPT = _B // _NS
_KI = _UPT // 128
_ROWS = _B // 128


def _ce_body(logits_ref, targets_ref, loss_ref):
    x = logits_ref[...]
    t = targets_ref[0, 0, :]
    m = jnp.max(x, axis=1)
    e = jnp.exp(x - m[:, None])
    s = jnp.sum(e, axis=1)
    cols = lax.broadcasted_iota(jnp.int32, (_R, _C), 1)
    tl = jnp.sum(jnp.where(cols == t[:, None], x, 0.0), axis=1)
    loss_ref[0, 0, :] = jnp.log(s) + m - tl


def _ce_loss(logits, targets):
    targets3 = targets.reshape(_NB, 1, _R)
    loss3 = pl.pallas_call(
        _ce_body,
        grid=(_NB,),
        in_specs=[
            pl.BlockSpec((_R, _C), lambda i: (i, 0)),
            pl.BlockSpec((1, 1, _R), lambda i: (i, 0, 0)),
        ],
        out_specs=pl.BlockSpec((1, 1, _R), lambda i: (i, 0, 0)),
        out_shape=jax.ShapeDtypeStruct((_NB, 1, _R), jnp.float32),
    )(logits, targets3)
    return loss3.reshape(_B)


def _drain(copies):
    for c in copies:
        c.wait()


def _ema_body(exp_hbm, idx_hbm, loss_hbm, dp_hbm, consts_hbm, ref_hbm,
              out_loss_hbm,
              idx_v, lidx2_v, pos_v, w_v, gath_v, loss_v, dp_v, nl_v, vv_v,
              consts_v, aux_sp, n_sp, sem):
    cid = lax.axis_index("c")
    sid = lax.axis_index("s")

    rowbase = sid * _KI
    c1 = pltpu.async_copy(idx_hbm.at[pl.ds(rowbase, _KI)], idx_v, sem)
    c2 = pltpu.async_copy(loss_hbm.at[pl.ds(rowbase, _KI)], loss_v, sem)
    c3 = pltpu.async_copy(dp_hbm.at[pl.ds(rowbase, _KI)], dp_v, sem)
    c4 = pltpu.async_copy(consts_hbm, consts_v, sem)
    _drain([c1, c2, c3, c4])
    _drain([pltpu.async_copy(exp_hbm.at[idx_v.at[j]], gath_v.at[j], sem)
            for j in range(_KI)])

    a = consts_v[0, :]
    c = consts_v[1, :]
    lane = lax.broadcasted_iota(jnp.int32, (16,), 0)
    for j in range(_KI):
        base = (rowbase + j) * 128
        for i in range(128 // 16):
            sl = pl.ds(i * 16, 16)
            g = gath_v[j, sl]
            l = loss_v[j, sl]
            n = g * _BETA + l * (1.0 - _BETA)
            gath_v[j, sl] = n
            nl_v[j, sl] = (n * a - c) / dp_v[j, sl]
            pos_v[j, sl] = base + i * 16 + lane

    _drain([pltpu.async_copy(gath_v.at[j],
                             n_sp.at[pl.ds((rowbase + j) * 128, 128)], sem)
            for j in range(_KI)])

    _drain([pltpu.async_copy(pos_v.at[j], aux_sp.at[idx_v.at[j]], sem)
            for j in range(_KI)])
    plsc.subcore_barrier()

    _drain([pltpu.async_copy(aux_sp.at[idx_v.at[j]], w_v.at[j], sem)
            for j in range(_KI)])
    for j in range(_KI):
        for i in range(128 // 16):
            sl = pl.ds(i * 16, 16)
            active = w_v[j, sl] < pos_v[j, sl]
            dummy = _M + (pos_v[j, sl] & (_NVIC - 1))
            lidx2_v[j, sl] = jnp.where(active, idx_v[j, sl], dummy)
    _drain([pltpu.async_copy(pos_v.at[j], aux_sp.at[lidx2_v.at[j]], sem)
            for j in range(_KI)])
    plsc.subcore_barrier()

    _drain([pltpu.async_copy(aux_sp.at[idx_v.at[j]], w_v.at[j], sem)
            for j in range(_KI)])
    _drain([pltpu.async_copy(n_sp.at[w_v.at[j]], vv_v.at[j], sem)
            for j in range(_KI)])
    _drain([pltpu.async_copy(vv_v.at[j], ref_hbm.at[idx_v.at[j]], sem)
            for j in range(_KI)])

    @pl.when(cid == 0)
    def _():
        pltpu.sync_copy(nl_v, out_loss_hbm.at[pl.ds(rowbase, _KI)])


def _ema_kernel():
    return pl.kernel(
        _ema_body,
        out_type=jax.ShapeDtypeStruct((_ROWS, 128), jnp.float32),
        mesh=plsc.VectorSubcoreMesh(core_axis_name="c", subcore_axis_name="s",
                                    num_cores=_NC, num_subcores=_NS),
        scratch_types=[
            pltpu.VMEM((_KI, 128), jnp.int32),
            pltpu.VMEM((_KI, 128), jnp.int32),
            pltpu.VMEM((_KI, 128), jnp.int32),
            pltpu.VMEM((_KI, 128), jnp.int32),
            pltpu.VMEM((_KI, 128), jnp.float32),
            pltpu.VMEM((_KI, 128), jnp.float32),
            pltpu.VMEM((_KI, 128), jnp.float32),
            pltpu.VMEM((_KI, 128), jnp.float32),
            pltpu.VMEM((_KI, 128), jnp.float32),
            pltpu.VMEM((2, 16), jnp.float32),
            pltpu.VMEM_SHARED((_SP,), jnp.int32),
            pltpu.VMEM_SHARED((_B,), jnp.float32),
            pltpu.SemaphoreType.DMA,
        ],
    )


def kernel(logits, targets, data_parameter_minibatch, exp_avg, index_dataset,
           epoch, switch):
    loss = _ce_loss(logits, targets.astype(jnp.int32))

    es = jnp.where(epoch < _SUPPRESSION_EPS,
                   (epoch + 1) / 10.0, 1.0).astype(jnp.float32)
    bias_cor = (1.0 - jnp.power(jnp.float32(_BETA),
                                (epoch + 1))).astype(jnp.float32)
    offset = jnp.where(switch != 0, _K1 * _GAMMA, _K1).astype(jnp.float32)
    a = es / bias_cor
    c = offset * es
    consts = jnp.stack([jnp.broadcast_to(a, (16,)), jnp.broadcast_to(c, (16,))])

    idx2 = index_dataset.astype(jnp.int32).reshape(_ROWS, 128)
    loss2 = loss.reshape(_ROWS, 128)
    dp2 = data_parameter_minibatch.reshape(_ROWS, 128)

    ref = jax.new_ref(exp_avg)
    new_loss2 = _ema_kernel()(exp_avg, idx2, loss2, dp2, consts, ref)
    return new_loss2.reshape(_B), ref[...]


# speedup vs baseline: 9.4927x; 1.0015x over previous
import jax
import jax.numpy as jnp
from jax import lax
from jax.experimental import pallas as pl
from jax.experimental.pallas import tpu as pltpu
from jax.experimental.pallas import tpu_sc as plsc

_B = 16384
_C = 1000
_M = 1000000
_BETA = 0.9
_GAMMA = 1.7
_SUPPRESSION_EPS = 10
_K1 = 10

_R = 4096
_NB = _B // _R

_NC = 2
_NS = 16
_NVIC = 2048
_SP = _M + _NVIC
_UPT = _B // _NS
_KI = _UPT // 128
_ROWS = _B // 128


def _ce_body(logits_ref, targets_ref, loss_ref):
    x = logits_ref[...]
    t = targets_ref[0, 0, :]
    m = jnp.max(x, axis=1)
    e = jnp.exp(x - m[:, None])
    s = jnp.sum(e, axis=1)
    cols = lax.broadcasted_iota(jnp.int32, (_R, _C), 1)
    tl = jnp.sum(jnp.where(cols == t[:, None], x, 0.0), axis=1)
    loss_ref[0, 0, :] = jnp.log(s) + m - tl


def _ce_loss(logits, targets):
    targets3 = targets.reshape(_NB, 1, _R)
    loss3 = pl.pallas_call(
        _ce_body,
        grid=(_NB,),
        in_specs=[
            pl.BlockSpec((_R, _C), lambda i: (i, 0)),
            pl.BlockSpec((1, 1, _R), lambda i: (i, 0, 0)),
        ],
        out_specs=pl.BlockSpec((1, 1, _R), lambda i: (i, 0, 0)),
        out_shape=jax.ShapeDtypeStruct((_NB, 1, _R), jnp.float32),
    )(logits, targets3)
    return loss3.reshape(_B)


def _drain(copies):
    for c in copies:
        c.wait()


def _ema_body(exp_hbm, idx_hbm, loss_hbm, dp_hbm, consts_hbm, ref_hbm,
              out_loss_hbm,
              idx_v, lidx2_v, pos_v, w_v, gath_v, loss_v, dp_v, nl_v, vv_v,
              consts_v, aux_sp, n_sp, sem):
    cid = lax.axis_index("c")
    sid = lax.axis_index("s")

    rowbase = sid * _KI
    c1 = pltpu.async_copy(idx_hbm.at[pl.ds(rowbase, _KI)], idx_v, sem)
    c2 = pltpu.async_copy(loss_hbm.at[pl.ds(rowbase, _KI)], loss_v, sem)
    c3 = pltpu.async_copy(dp_hbm.at[pl.ds(rowbase, _KI)], dp_v, sem)
    c4 = pltpu.async_copy(consts_hbm, consts_v, sem)
    _drain([c1, c2, c3, c4])
    _drain([pltpu.async_copy(exp_hbm.at[idx_v.at[j]], gath_v.at[j], sem)
            for j in range(_KI)])

    a = consts_v[0, :]
    c = consts_v[1, :]
    lane = lax.broadcasted_iota(jnp.int32, (16,), 0)
    for j in range(_KI):
        base = (rowbase + j) * 128
        for i in range(128 // 16):
            sl = pl.ds(i * 16, 16)
            g = gath_v[j, sl]
            l = loss_v[j, sl]
            n = g * _BETA + l * (1.0 - _BETA)
            gath_v[j, sl] = n
            nl_v[j, sl] = (n * a - c) / dp_v[j, sl]
            pos_v[j, sl] = base + i * 16 + lane

    _drain([pltpu.async_copy(gath_v.at[j],
                             n_sp.at[pl.ds((rowbase + j) * 128, 128)], sem)
            for j in range(_KI)])

    _drain([pltpu.async_copy(pos_v.at[j], aux_sp.at[idx_v.at[j]], sem)
            for j in range(_KI)])
    plsc.subcore_barrier()

    _drain([pltpu.async_copy(aux_sp.at[idx_v.at[j]], w_v.at[j], sem)
            for j in range(_KI)])
    for j in range(_KI):
        for i in range(128 // 16):
            sl = pl.ds(i * 16, 16)
            active = w_v[j, sl] < pos_v[j, sl]
            dummy = _M + (pos_v[j, sl] & (_NVIC - 1))
            lidx2_v[j, sl] = jnp.where(active, idx_v[j, sl], dummy)
    _drain([pltpu.async_copy(pos_v.at[j], aux_sp.at[lidx2_v.at[j]], sem)
            for j in range(_KI)])
    plsc.subcore_barrier()

    _drain([pltpu.async_copy(aux_sp.at[idx_v.at[j]], w_v.at[j], sem)
            for j in range(_KI)])
    _drain([pltpu.async_copy(n_sp.at[w_v.at[j]], vv_v.at[j], sem)
            for j in range(_KI)])
    _drain([pltpu.async_copy(vv_v.at[j], ref_hbm.at[idx_v.at[j]], sem)
            for j in range(_KI)])

    @pl.when(cid == 0)
    def _():
        pltpu.sync_copy(nl_v, out_loss_hbm.at[pl.ds(rowbase, _KI)])


def _ema_kernel():
    return pl.kernel(
        _ema_body,
        out_type=jax.ShapeDtypeStruct((_ROWS, 128), jnp.float32),
        mesh=plsc.VectorSubcoreMesh(core_axis_name="c", subcore_axis_name="s",
                                    num_cores=_NC, num_subcores=_NS),
        scratch_types=[
            pltpu.VMEM((_KI, 128), jnp.int32),
            pltpu.VMEM((_KI, 128), jnp.int32),
            pltpu.VMEM((_KI, 128), jnp.int32),
            pltpu.VMEM((_KI, 128), jnp.int32),
            pltpu.VMEM((_KI, 128), jnp.float32),
            pltpu.VMEM((_KI, 128), jnp.float32),
            pltpu.VMEM((_KI, 128), jnp.float32),
            pltpu.VMEM((_KI, 128), jnp.float32),
            pltpu.VMEM((_KI, 128), jnp.float32),
            pltpu.VMEM((2, 16), jnp.float32),
            pltpu.VMEM_SHARED((_SP,), jnp.int32),
            pltpu.VMEM_SHARED((_B,), jnp.float32),
            pltpu.SemaphoreType.DMA,
        ],
    )


def kernel(logits, targets, data_parameter_minibatch, exp_avg, index_dataset,
           epoch, switch):
    loss = _ce_loss(logits, targets.astype(jnp.int32))

    es = jnp.where(epoch < _SUPPRESSION_EPS,
                   (epoch + 1) / 10.0, 1.0).astype(jnp.float32)
    bias_cor = (1.0 - jnp.power(jnp.float32(_BETA),
                                (epoch + 1))).astype(jnp.float32)
    offset = jnp.where(switch != 0, _K1 * _GAMMA, _K1).astype(jnp.float32)
    a = es / bias_cor
    c = offset * es
    consts = jnp.stack([jnp.broadcast_to(a, (16,)), jnp.broadcast_to(c, (16,))])

    idx2 = index_dataset.astype(jnp.int32).reshape(_ROWS, 128)
    loss2 = loss.reshape(_ROWS, 128)
    dp2 = data_parameter_minibatch.reshape(_ROWS, 128)

    ref = jax.new_ref(exp_avg)
    new_loss2 = _ema_kernel()(exp_avg, idx2, loss2, dp2, consts, ref)
    return new_loss2.reshape(_B), ref[...]
